# Initial kernel scaffold; baseline (speedup 1.0000x reference)
import jax
import jax.numpy as jnp
from jax.experimental import pallas as pl


def kernel(heads, relations, center_embedding, offset_embedding, center_mul, center_add, offset_mul, offset_add):
    raise NotImplementedError("write your pallas kernel here")


# speedup vs baseline: 4.0107x; 4.0107x over previous
import functools

import jax
import jax.numpy as jnp
from jax import lax
from jax.experimental import pallas as pl
from jax.experimental.pallas import tpu as pltpu
from jax.experimental.pallas import tpu_sc as plsc

BATCH = 16384
DIM = 128
LANES = 16
CHUNK = 128


def _body(heads_hbm, rel_hbm, ctr_hbm, off_hbm, cm_hbm, ca_hbm, om_hbm,
          oa_hbm, outc_hbm, outo_hbm,
          hidx_v, ridx_v, c_v, o_v, cm_v, ca_v, om_v, oa_v, sem,
          *, b_per_w, num_cores):
    wid = lax.axis_index("s") * num_cores + lax.axis_index("c")
    base = wid * b_per_w

    pltpu.sync_copy(heads_hbm.at[pl.ds(base, b_per_w)], hidx_v)
    pltpu.sync_copy(rel_hbm.at[pl.ds(base, b_per_w)], ridx_v)

    n_chunks = b_per_w // CHUNK
    groups = DIM // LANES

    for k in range(n_chunks):
        hs = hidx_v.at[pl.ds(k * CHUNK, CHUNK)]
        rs = ridx_v.at[pl.ds(k * CHUNK, CHUNK)]
        cps = [
            pltpu.async_copy(ctr_hbm.at[hs], c_v, sem),
            pltpu.async_copy(off_hbm.at[hs], o_v, sem),
            pltpu.async_copy(cm_hbm.at[rs], cm_v, sem),
            pltpu.async_copy(ca_hbm.at[rs], ca_v, sem),
            pltpu.async_copy(om_hbm.at[rs], om_v, sem),
            pltpu.async_copy(oa_hbm.at[rs], oa_v, sem),
        ]
        for cp in cps:
            cp.wait()

        def row(r, _):
            for j in range(groups):
                sl = pl.ds(j * LANES, LANES)
                c = c_v[r, sl]
                cm = cm_v[r, sl]
                ca = ca_v[r, sl]
                c_v[r, sl] = c * cm + ca
                o = jnp.abs(o_v[r, sl])
                om = jnp.abs(om_v[r, sl])
                oa = jnp.abs(oa_v[r, sl])
                o_v[r, sl] = o * om + oa
            return _

        lax.fori_loop(0, CHUNK, row, None)

        out_sl = pl.ds(base + k * CHUNK, CHUNK)
        pltpu.sync_copy(c_v, outc_hbm.at[out_sl])
        pltpu.sync_copy(o_v, outo_hbm.at[out_sl])


def kernel(heads, relations, center_embedding, offset_embedding,
           center_mul, center_add, offset_mul, offset_add):
    info = plsc.get_sparse_core_info()
    nw = info.num_cores * info.num_subcores
    b_per_w = BATCH // nw

    mesh = plsc.VectorSubcoreMesh(core_axis_name="c", subcore_axis_name="s")
    out_t = jax.ShapeDtypeStruct((BATCH, DIM), jnp.float32)

    run = pl.kernel(
        functools.partial(_body, b_per_w=b_per_w, num_cores=info.num_cores),
        mesh=mesh,
        out_type=(out_t, out_t),
        scratch_types=[
            pltpu.VMEM((b_per_w,), jnp.int32),
            pltpu.VMEM((b_per_w,), jnp.int32),
            pltpu.VMEM((CHUNK, DIM), jnp.float32),
            pltpu.VMEM((CHUNK, DIM), jnp.float32),
            pltpu.VMEM((CHUNK, DIM), jnp.float32),
            pltpu.VMEM((CHUNK, DIM), jnp.float32),
            pltpu.VMEM((CHUNK, DIM), jnp.float32),
            pltpu.VMEM((CHUNK, DIM), jnp.float32),
            pltpu.SemaphoreType.DMA,
        ],
    )
    return run(heads.astype(jnp.int32), relations.astype(jnp.int32),
               center_embedding, offset_embedding,
               center_mul, center_add, offset_mul, offset_add)


# speedup vs baseline: 4.8927x; 1.2199x over previous
import functools

import jax
import jax.numpy as jnp
from jax import lax
from jax.experimental import pallas as pl
from jax.experimental.pallas import tpu as pltpu
from jax.experimental.pallas import tpu_sc as plsc

BATCH = 16384
DIM = 128
LANES = 16
CHUNK = 64
GROUPS = DIM // LANES


def _body(heads_hbm, rel_hbm, ctr_hbm, off_hbm, cm_hbm, ca_hbm, om_hbm,
          oa_hbm, outc_hbm, outo_hbm,
          hidx_v, ridx_v, bufs0, bufs1,
          gsem0, gsem1, osem0, osem1,
          *, b_per_w, num_cores):
    wid = lax.axis_index("s") * num_cores + lax.axis_index("c")
    base = wid * b_per_w
    bufs = (bufs0, bufs1)
    gsem = (gsem0, gsem1)
    osem = (osem0, osem1)

    pltpu.sync_copy(heads_hbm.at[pl.ds(base, b_per_w)], hidx_v)
    pltpu.sync_copy(rel_hbm.at[pl.ds(base, b_per_w)], ridx_v)

    n_chunks = b_per_w // CHUNK

    def fire(k, b):
        hs = hidx_v.at[pl.ds(k * CHUNK, CHUNK)]
        rs = ridx_v.at[pl.ds(k * CHUNK, CHUNK)]
        bb = bufs[b]
        return [
            pltpu.async_copy(ctr_hbm.at[hs], bb.at[0], gsem[b]),
            pltpu.async_copy(off_hbm.at[hs], bb.at[1], gsem[b]),
            pltpu.async_copy(cm_hbm.at[rs], bb.at[2], gsem[b]),
            pltpu.async_copy(om_hbm.at[rs], bb.at[3], gsem[b]),
            pltpu.async_copy(ca_hbm.at[rs], bb.at[4], gsem[b]),
            pltpu.async_copy(oa_hbm.at[rs], bb.at[5], gsem[b]),
        ]

    def compute(b):
        bb = bufs[b]

        def row(r, _):
            for j in range(GROUPS):
                sl = pl.ds(j * LANES, LANES)
                plsc.addupdate(bb.at[4, r, sl], bb[0, r, sl] * bb[2, r, sl])
                bb[5, r, sl] = (jnp.abs(bb[1, r, sl] * bb[3, r, sl])
                                + jnp.abs(bb[5, r, sl]))
            return _

        lax.fori_loop(0, CHUNK, row, None)

    cps = {0: fire(0, 0)}
    outs = {0: [], 1: []}
    for k in range(n_chunks):
        b = k % 2
        if k + 1 < n_chunks:
            nb = (k + 1) % 2
            for d in outs[nb]:
                d.wait()
            outs[nb] = []
            cps[k + 1] = fire(k + 1, nb)
        for d in cps.pop(k):
            d.wait()
        compute(b)
        out_sl = pl.ds(base + k * CHUNK, CHUNK)
        outs[b] = [
            pltpu.async_copy(bufs[b].at[4], outc_hbm.at[out_sl], osem[b]),
            pltpu.async_copy(bufs[b].at[5], outo_hbm.at[out_sl], osem[b]),
        ]
    for b in (0, 1):
        for d in outs[b]:
            d.wait()


def kernel(heads, relations, center_embedding, offset_embedding,
           center_mul, center_add, offset_mul, offset_add):
    info = plsc.get_sparse_core_info()
    nw = info.num_cores * info.num_subcores
    b_per_w = BATCH // nw

    mesh = plsc.VectorSubcoreMesh(core_axis_name="c", subcore_axis_name="s")
    out_t = jax.ShapeDtypeStruct((BATCH, DIM), jnp.float32)
    buf_t = pltpu.VMEM((6, CHUNK, DIM), jnp.float32)

    run = pl.kernel(
        functools.partial(_body, b_per_w=b_per_w, num_cores=info.num_cores),
        mesh=mesh,
        out_type=(out_t, out_t),
        scratch_types=[
            pltpu.VMEM((b_per_w,), jnp.int32),
            pltpu.VMEM((b_per_w,), jnp.int32),
            buf_t, buf_t,
            pltpu.SemaphoreType.DMA, pltpu.SemaphoreType.DMA,
            pltpu.SemaphoreType.DMA, pltpu.SemaphoreType.DMA,
        ],
    )
    return run(heads.astype(jnp.int32), relations.astype(jnp.int32),
               center_embedding, offset_embedding,
               center_mul, center_add, offset_mul, offset_add)
